# Initial kernel scaffold; baseline (speedup 1.0000x reference)
import jax
import jax.numpy as jnp
from jax.experimental import pallas as pl


def kernel(x, table):
    raise NotImplementedError("write your pallas kernel here")


# speedup vs baseline: 1.0939x; 1.0939x over previous
import functools

import jax
import jax.numpy as jnp
from jax import lax
from jax.experimental import pallas as pl
from jax.experimental.pallas import tpu as pltpu
from jax.experimental.pallas import tpu_sc as plsc

_D = 32
_IPS = 128
_K = 8
_CHUNK = _K * _IPS
_NC = 2
_NS = 16
_NW = _NC * _NS


@functools.lru_cache(maxsize=None)
def _build(n_rows: int):
    per_w = n_rows // _NW
    n_chunks = per_w // _CHUNK
    assert per_w * _NW == n_rows and n_chunks * _CHUNK == per_w

    mesh = plsc.VectorSubcoreMesh(core_axis_name="c", subcore_axis_name="s")

    @functools.partial(
        pl.kernel,
        mesh=mesh,
        compiler_params=pltpu.CompilerParams(use_tc_tiling_on_sc=False),
        out_type=jax.ShapeDtypeStruct((n_rows, _D), jnp.float32),
        scratch_types=[
            pltpu.VMEM((_K, _IPS), jnp.int32),
            pltpu.VMEM((_CHUNK, _D), jnp.float32),
            pltpu.SemaphoreType.DMA,
        ],
    )
    def gather_kernel(table_hbm, idx_hbm, out_hbm, idx_v, rows_v, sem):
        wid = lax.axis_index("s") * _NC + lax.axis_index("c")

        def chunk_body(j, carry):
            pltpu.sync_copy(idx_hbm.at[wid, j], idx_v)
            descs = [
                pltpu.async_copy(
                    table_hbm.at[idx_v.at[b]],
                    rows_v.at[pl.ds(b * _IPS, _IPS)],
                    sem,
                )
                for b in range(_K)
            ]
            for d in descs:
                d.wait()
            base = (wid * n_chunks + j) * _CHUNK
            pltpu.sync_copy(rows_v, out_hbm.at[pl.ds(base, _CHUNK)])
            return carry

        lax.fori_loop(0, n_chunks, chunk_body, 0)

    return gather_kernel


def kernel(x, table):
    b, h = x.shape
    n_rows = b * h
    idx = x.astype(jnp.int32).reshape(_NW, n_rows // (_NW * _CHUNK), _K, _IPS)
    out = _build(n_rows)(table, idx)
    return out.reshape(b, h, _D)


# speedup vs baseline: 1.1115x; 1.0161x over previous
import functools

import jax
import jax.numpy as jnp
from jax import lax
from jax.experimental import pallas as pl
from jax.experimental.pallas import tpu as pltpu
from jax.experimental.pallas import tpu_sc as plsc

_D = 32
_IPS = 128
_K = 10
_CHUNK = _K * _IPS
_NC = 2
_NS = 16
_NW = _NC * _NS


@functools.lru_cache(maxsize=None)
def _build(n_rows: int):
    per_w = n_rows // _NW
    n_chunks = per_w // _CHUNK
    assert per_w * _NW == n_rows and n_chunks * _CHUNK == per_w
    assert n_chunks % 2 == 0
    n_pairs = n_chunks // 2

    mesh = plsc.VectorSubcoreMesh(core_axis_name="c", subcore_axis_name="s")

    @functools.partial(
        pl.kernel,
        mesh=mesh,
        compiler_params=pltpu.CompilerParams(use_tc_tiling_on_sc=False),
        out_type=jax.ShapeDtypeStruct((n_rows, _D), jnp.float32),
        scratch_types=[
            pltpu.VMEM((_K, _IPS), jnp.int32),
            pltpu.VMEM((_K, _IPS), jnp.int32),
            pltpu.VMEM((_CHUNK, _D), jnp.float32),
            pltpu.VMEM((_CHUNK, _D), jnp.float32),
            pltpu.SemaphoreType.DMA,
            pltpu.SemaphoreType.DMA,
            pltpu.SemaphoreType.DMA,
            pltpu.SemaphoreType.DMA,
        ],
    )
    def gather_kernel(table_hbm, idx_hbm, out_hbm, idx_a, idx_b,
                      rows_a, rows_b, gsem_a, gsem_b, ssem_a, ssem_b):
        wid = lax.axis_index("s") * _NC + lax.axis_index("c")

        def fire(j, idx_v, rows_v, gsem):
            pltpu.sync_copy(idx_hbm.at[wid, j], idx_v)
            for b in range(_K):
                pltpu.async_copy(
                    table_hbm.at[idx_v.at[b]],
                    rows_v.at[pl.ds(b * _IPS, _IPS)],
                    gsem,
                )

        def drain_gathers(rows_v, gsem):
            pltpu.make_async_copy(
                out_hbm.at[pl.ds(0, _CHUNK)], rows_v, gsem).wait()

        def store(j, rows_v, ssem):
            base = (wid * n_chunks + j) * _CHUNK
            pltpu.async_copy(rows_v, out_hbm.at[pl.ds(base, _CHUNK)], ssem)

        def wait_store(rows_v, ssem):
            pltpu.make_async_copy(
                rows_v, out_hbm.at[pl.ds(0, _CHUNK)], ssem).wait()

        def pair_body(t, carry):
            j0 = 2 * t
            j1 = j0 + 1

            @pl.when(t > 0)
            def _():
                wait_store(rows_a, ssem_a)

            fire(j0, idx_a, rows_a, gsem_a)

            @pl.when(t > 0)
            def _():
                wait_store(rows_b, ssem_b)

            fire(j1, idx_b, rows_b, gsem_b)

            drain_gathers(rows_a, gsem_a)
            store(j0, rows_a, ssem_a)
            drain_gathers(rows_b, gsem_b)
            store(j1, rows_b, ssem_b)
            return carry

        lax.fori_loop(0, n_pairs, pair_body, 0)
        wait_store(rows_a, ssem_a)
        wait_store(rows_b, ssem_b)

    return gather_kernel


def kernel(x, table):
    b, h = x.shape
    n_rows = b * h
    idx = x.astype(jnp.int32).reshape(_NW, n_rows // (_NW * _CHUNK), _K, _IPS)
    out = _build(n_rows)(table, idx)
    return out.reshape(b, h, _D)


# speedup vs baseline: 1.7995x; 1.6189x over previous
import functools

import jax
import jax.numpy as jnp
from jax import lax
from jax.experimental import pallas as pl
from jax.experimental.pallas import tpu as pltpu
from jax.experimental.pallas import tpu_sc as plsc

_D = 32
_RPC = 16
_NC = 2
_NS = 16
_NW = _NC * _NS


@functools.lru_cache(maxsize=None)
def _build(batch: int, hist: int):
    rows_per_w = batch // _NW
    n_chunks = rows_per_w // _RPC
    assert rows_per_w * _NW == batch and n_chunks * _RPC == rows_per_w
    assert n_chunks % 2 == 0
    n_pairs = n_chunks // 2

    mesh = plsc.VectorSubcoreMesh(core_axis_name="c", subcore_axis_name="s")

    @functools.partial(
        pl.kernel,
        mesh=mesh,
        compiler_params=pltpu.CompilerParams(use_tc_tiling_on_sc=False),
        out_type=jax.ShapeDtypeStruct((batch, hist, _D), jnp.float32),
        scratch_types=[
            pltpu.VMEM((_RPC, hist), jnp.int32),
            pltpu.VMEM((_RPC, hist), jnp.int32),
            pltpu.VMEM((_RPC, hist, _D), jnp.float32),
            pltpu.VMEM((_RPC, hist, _D), jnp.float32),
            pltpu.SemaphoreType.DMA,
            pltpu.SemaphoreType.DMA,
            pltpu.SemaphoreType.DMA,
            pltpu.SemaphoreType.DMA,
        ],
    )
    def gather_kernel(table_hbm, x_hbm, out_hbm, idx_a, idx_b,
                      rows_a, rows_b, gsem_a, gsem_b, ssem_a, ssem_b):
        wid = lax.axis_index("s") * _NC + lax.axis_index("c")

        def fire(j, idx_v, rows_v, gsem):
            row0 = wid * rows_per_w + j * _RPC
            pltpu.sync_copy(x_hbm.at[pl.ds(row0, _RPC)], idx_v)
            for r in range(_RPC):
                pltpu.async_copy(
                    table_hbm.at[idx_v.at[r]], rows_v.at[r], gsem)

        def drain_gathers(rows_v, gsem):
            pltpu.make_async_copy(
                out_hbm.at[pl.ds(0, _RPC)], rows_v, gsem).wait()

        def store(j, rows_v, ssem):
            row0 = wid * rows_per_w + j * _RPC
            pltpu.async_copy(rows_v, out_hbm.at[pl.ds(row0, _RPC)], ssem)

        def wait_store(rows_v, ssem):
            pltpu.make_async_copy(
                rows_v, out_hbm.at[pl.ds(0, _RPC)], ssem).wait()

        def pair_body(t, carry):
            j0 = 2 * t
            j1 = j0 + 1

            @pl.when(t > 0)
            def _():
                wait_store(rows_a, ssem_a)

            fire(j0, idx_a, rows_a, gsem_a)

            @pl.when(t > 0)
            def _():
                wait_store(rows_b, ssem_b)

            fire(j1, idx_b, rows_b, gsem_b)

            drain_gathers(rows_a, gsem_a)
            store(j0, rows_a, ssem_a)
            drain_gathers(rows_b, gsem_b)
            store(j1, rows_b, ssem_b)
            return carry

        lax.fori_loop(0, n_pairs, pair_body, 0)
        wait_store(rows_a, ssem_a)
        wait_store(rows_b, ssem_b)

    return gather_kernel


def kernel(x, table):
    b, h = x.shape
    return _build(b, h)(table, x.astype(jnp.int32))
